# baseline (speedup 1.0000x reference)
import jax
import jax.numpy as jnp
from jax import lax
from jax.experimental import pallas as pl
from jax.experimental.pallas import tpu as pltpu
from jax.experimental.pallas import tpu_sc as plsc

L = 20
B = 16384
N_STATES = 2 ** L

_NC = 2
_NS = 16
_NW = _NC * _NS
_BPW = B // _NW
_NCHUNK = _BPW // 128

_TC_GRID = 2
_TC_COLS = B // _TC_GRID


def _tc_idx_body(xt_ref, idx_ref):
    x = xt_ref[...]
    shift = (L - 1) - lax.broadcasted_iota(jnp.int32, (L, _TC_COLS), 0)
    w = lax.shift_left(1, shift).astype(jnp.float32)
    s = jnp.sum(x * w, axis=0) + jnp.float32(2 ** L - 1)
    idx_ref[...] = (s * 0.5).astype(jnp.int32)


def _sc_gather_body(idx_hbm, table_hbm, out_hbm, idxv, outv, g0, g1, g2, g3, wsem):
    wid = lax.axis_index("s") * _NC + lax.axis_index("c")
    base = wid * _BPW
    gsems = (g0, g1, g2, g3)

    pltpu.sync_copy(idx_hbm.at[pl.ds(base, _BPW)], idxv)

    copies = [
        pltpu.async_copy(
            table_hbm.at[idxv.at[pl.ds(c * 128, 128)]],
            outv.at[pl.ds(c * 128, 128)],
            gsems[c],
        )
        for c in range(_NCHUNK)
    ]
    writes = []
    for c in range(_NCHUNK):
        copies[c].wait()
        writes.append(
            pltpu.async_copy(
                outv.at[pl.ds(c * 128, 128)],
                out_hbm.at[pl.ds(base + c * 128, 128)],
                wsem,
            )
        )
    for w in writes:
        w.wait()


@jax.jit
def kernel(x_in, logstate):
    idx = pl.pallas_call(
        _tc_idx_body,
        grid=(_TC_GRID,),
        in_specs=[pl.BlockSpec((L, _TC_COLS), lambda i: (0, i))],
        out_specs=pl.BlockSpec((_TC_COLS,), lambda i: (i,)),
        out_shape=jax.ShapeDtypeStruct((B,), jnp.int32),
    )(x_in.T)

    mesh = plsc.VectorSubcoreMesh(core_axis_name="c", subcore_axis_name="s")
    run = pl.kernel(
        _sc_gather_body,
        mesh=mesh,
        out_type=jax.ShapeDtypeStruct((B,), jnp.float32),
        scratch_types=[
            pltpu.VMEM((_BPW,), jnp.int32),
            pltpu.VMEM((_BPW,), jnp.float32),
            pltpu.SemaphoreType.DMA,
            pltpu.SemaphoreType.DMA,
            pltpu.SemaphoreType.DMA,
            pltpu.SemaphoreType.DMA,
            pltpu.SemaphoreType.DMA,
        ],
        compiler_params=pltpu.CompilerParams(needs_layout_passes=False),
    )
    return run(idx, logstate)


# speedup vs baseline: 1.0019x; 1.0019x over previous
import jax
import jax.numpy as jnp
from jax import lax
from jax.experimental import pallas as pl
from jax.experimental.pallas import tpu as pltpu
from jax.experimental.pallas import tpu_sc as plsc

L = 20
B = 16384
N_STATES = 2 ** L

_NC = 2
_NS = 16
_NW = _NC * _NS
_BPW = B // _NW
_NCHUNK = _BPW // 128

_TC_GRID = 1
_TC_COLS = B // _TC_GRID


def _tc_idx_body(xt_ref, idx_ref):
    x = xt_ref[...]
    shift = (L - 1) - lax.broadcasted_iota(jnp.int32, (L, _TC_COLS), 0)
    w = lax.shift_left(1, shift).astype(jnp.float32)
    s = jnp.sum(x * w, axis=0) + jnp.float32(2 ** L - 1)
    idx_ref[...] = (s * 0.5).astype(jnp.int32)


def _sc_gather_body(idx_hbm, table_hbm, out_hbm, idxv, outv, g0, g1, g2, g3, wsem):
    wid = lax.axis_index("s") * _NC + lax.axis_index("c")
    base = wid * _BPW
    gsems = (g0, g1, g2, g3)

    half = _BPW // 2
    stg_a = pltpu.async_copy(
        idx_hbm.at[pl.ds(base, half)], idxv.at[pl.ds(0, half)], wsem
    )
    stg_b = pltpu.async_copy(
        idx_hbm.at[pl.ds(base + half, half)], idxv.at[pl.ds(half, half)], g3
    )

    copies = [None] * _NCHUNK
    stg_a.wait()
    for c in (0, 1):
        copies[c] = pltpu.async_copy(
            table_hbm.at[idxv.at[pl.ds(c * 128, 128)]],
            outv.at[pl.ds(c * 128, 128)],
            gsems[c],
        )
    stg_b.wait()
    for c in (2, 3):
        copies[c] = pltpu.async_copy(
            table_hbm.at[idxv.at[pl.ds(c * 128, 128)]],
            outv.at[pl.ds(c * 128, 128)],
            gsems[c],
        )
    writes = []
    for c in range(_NCHUNK):
        copies[c].wait()
        writes.append(
            pltpu.async_copy(
                outv.at[pl.ds(c * 128, 128)],
                out_hbm.at[pl.ds(base + c * 128, 128)],
                wsem,
            )
        )
    for w in writes:
        w.wait()


@jax.jit
def kernel(x_in, logstate):
    idx = pl.pallas_call(
        _tc_idx_body,
        grid=(_TC_GRID,),
        in_specs=[pl.BlockSpec((L, _TC_COLS), lambda i: (0, i))],
        out_specs=pl.BlockSpec((_TC_COLS,), lambda i: (i,)),
        out_shape=jax.ShapeDtypeStruct((B,), jnp.int32),
    )(x_in.T)

    mesh = plsc.VectorSubcoreMesh(core_axis_name="c", subcore_axis_name="s")
    run = pl.kernel(
        _sc_gather_body,
        mesh=mesh,
        out_type=jax.ShapeDtypeStruct((B,), jnp.float32),
        scratch_types=[
            pltpu.VMEM((_BPW,), jnp.int32),
            pltpu.VMEM((_BPW,), jnp.float32),
            pltpu.SemaphoreType.DMA,
            pltpu.SemaphoreType.DMA,
            pltpu.SemaphoreType.DMA,
            pltpu.SemaphoreType.DMA,
            pltpu.SemaphoreType.DMA,
        ],
        compiler_params=pltpu.CompilerParams(needs_layout_passes=False),
    )
    return run(idx, logstate)
